# Initial kernel scaffold; baseline (speedup 1.0000x reference)
import functools

import jax
import jax.numpy as jnp
from jax import lax
from jax.experimental import pallas as pl
from jax.experimental.pallas import tpu as pltpu
from jax.experimental.pallas import tpu_sc as plsc

N = 10000
E = 320000
F_IN = 128
G = 64
THRESH = 0.5

NPAD = 10240
K = 128
NCHUNK = E // K
NW = 32
CPT = -(-NCHUNK // NW)
ROWS_PER_TILE = NPAD // 16

_MESH = plsc.VectorSubcoreMesh(core_axis_name="c", subcore_axis_name="s")
_HIGH = lax.Precision.HIGHEST


def _worker_id():
    return lax.axis_index("s") * 2 + lax.axis_index("c")


@functools.partial(
    pl.kernel,
    out_type=jax.ShapeDtypeStruct((2, NPAD), jnp.float32),
    mesh=_MESH,
    scratch_types=[
        pltpu.VMEM((K,), jnp.int32),
        pltpu.VMEM((K,), jnp.float32),
        pltpu.VMEM((K,), jnp.float32),
        pltpu.VMEM_SHARED((NPAD,), jnp.float32),
    ],
)
def _deg_kernel(col_hbm, attr_hbm, out_hbm, col_v, w_v, zero_v, acc_sh):
    c = lax.axis_index("c")
    s = lax.axis_index("s")
    wid = _worker_id()

    for j in range(K // 16):
        zero_v[pl.ds(j * 16, 16)] = jnp.zeros((16,), jnp.float32)
    for i in range(ROWS_PER_TILE // K):
        pltpu.sync_copy(zero_v, acc_sh.at[pl.ds(s * ROWS_PER_TILE + i * K, K)])
    plsc.subcore_barrier()

    def body(i, carry):
        chunk = i * NW + wid

        @pl.when(chunk < NCHUNK)
        def _():
            base = chunk * K
            pltpu.sync_copy(col_hbm.at[pl.ds(base, K)], col_v)
            pltpu.sync_copy(attr_hbm.at[pl.ds(base, K)], w_v)
            for j in range(K // 16):
                a = w_v[pl.ds(j * 16, 16)]
                w_v[pl.ds(j * 16, 16)] = jnp.where(a > THRESH, a, 0.0)
            pltpu.sync_copy(w_v, acc_sh.at[col_v], add=True)

        return carry

    lax.fori_loop(0, CPT, body, 0)
    plsc.subcore_barrier()
    pltpu.sync_copy(acc_sh.at[pl.ds(s * ROWS_PER_TILE, ROWS_PER_TILE)],
                    out_hbm.at[c, pl.ds(s * ROWS_PER_TILE, ROWS_PER_TILE)])


def _make_prop(F):
    @functools.partial(
        pl.kernel,
        out_type=jax.ShapeDtypeStruct((2, NPAD, F), jnp.float32),
        mesh=_MESH,
        scratch_types=[
            pltpu.VMEM((K,), jnp.int32),
            pltpu.VMEM((K,), jnp.int32),
            pltpu.VMEM((K,), jnp.float32),
            pltpu.VMEM((K, F), jnp.float32),
            pltpu.VMEM_SHARED((NPAD, F), jnp.float32),
            pltpu.SemaphoreType.DMA,
        ],
    )
    def _prop(row_hbm, col_hbm, attr_hbm, u_hbm, out_hbm,
              row_v, col_v, w_v, msg_v, acc_sh, sem):
        c = lax.axis_index("c")
        s = lax.axis_index("s")
        wid = _worker_id()

        def zrow(j, carry):
            for f in range(F // 16):
                msg_v[j, pl.ds(f * 16, 16)] = jnp.zeros((16,), jnp.float32)
            return carry

        lax.fori_loop(0, K, zrow, 0)
        for i in range(ROWS_PER_TILE // K):
            pltpu.sync_copy(
                msg_v, acc_sh.at[pl.ds(s * ROWS_PER_TILE + i * K, K)])
        plsc.subcore_barrier()

        def body(i, carry):
            chunk = i * NW + wid

            @pl.when(chunk < NCHUNK)
            def _():
                base = chunk * K
                pltpu.sync_copy(row_hbm.at[pl.ds(base, K)], row_v)
                pltpu.sync_copy(col_hbm.at[pl.ds(base, K)], col_v)
                pltpu.sync_copy(attr_hbm.at[pl.ds(base, K)], w_v)
                for j in range(K // 16):
                    a = w_v[pl.ds(j * 16, 16)]
                    w_v[pl.ds(j * 16, 16)] = jnp.where(a > THRESH, a, 0.0)
                pltpu.async_copy(u_hbm.at[row_v], msg_v, sem).wait()

                def scale(jj, carry2):
                    wsp = plsc.load_gather(
                        w_v, [jnp.full((16,), jj, jnp.int32)])
                    for f in range(F // 16):
                        sl = pl.ds(f * 16, 16)
                        msg_v[jj, sl] = msg_v[jj, sl] * wsp
                    return carry2

                lax.fori_loop(0, K, scale, 0)
                pltpu.sync_copy(msg_v, acc_sh.at[col_v], add=True)

            return carry

        lax.fori_loop(0, CPT, body, 0)
        plsc.subcore_barrier()
        pltpu.sync_copy(
            acc_sh.at[pl.ds(s * ROWS_PER_TILE, ROWS_PER_TILE)],
            out_hbm.at[c, pl.ds(s * ROWS_PER_TILE, ROWS_PER_TILE)])

    return _prop


_prop64 = _make_prop(64)
_prop128 = _make_prop(128)


def _tc1_body(x_ref, w1_ref, degp_ref, u1_ref, dis_ref):
    degp = degp_ref[...]
    deg = degp[0, :N] + degp[1, :N] + 1.0
    dis = jnp.where(deg > 0.0, lax.rsqrt(deg), 0.0)
    xw = lax.dot_general(x_ref[...], w1_ref[...],
                         (((1,), (0,)), ((), ())),
                         precision=_HIGH, preferred_element_type=jnp.float32)
    u1_ref[...] = xw * dis
    dis_ref[...] = dis


def _bn_lrelu(h, g, be):
    mean = jnp.mean(h, axis=0, keepdims=True)
    var = jnp.mean((h - mean) * (h - mean), axis=0, keepdims=True)
    hn = g * (h - mean) / jnp.sqrt(var + 1e-5) + be
    return jnp.where(hn >= 0.0, hn, 0.01 * hn)


def _tc2_body(sp_ref, u1_ref, dis_ref, b1_ref, g1_ref, be1_ref, w2_ref,
              u2_ref):
    sp = sp_ref[...]
    dis = dis_ref[...]
    u1 = u1_ref[...]
    h = dis * (sp[0, :N] + sp[1, :N] + u1) + b1_ref[...]
    h = _bn_lrelu(h, g1_ref[...], be1_ref[...])
    xw = lax.dot_general(h, w2_ref[...], (((1,), (0,)), ((), ())),
                         precision=_HIGH, preferred_element_type=jnp.float32)
    u2_ref[...] = xw * dis


def _tc3_body(sp_ref, u2_ref, dis_ref, b2_ref, g2_ref, be2_ref, batch_ref,
              wout_ref, bout_ref, out_ref):
    sp = sp_ref[...]
    dis = dis_ref[...]
    u2 = u2_ref[...]
    h = dis * (sp[0, :N] + sp[1, :N] + u2) + b2_ref[...]
    h = _bn_lrelu(h, g2_ref[...], be2_ref[...])
    onehot = (batch_ref[...] ==
              lax.broadcasted_iota(jnp.int32, (1, G), 1)).astype(jnp.float32)
    sums = lax.dot_general(onehot, h, (((0,), (0,)), ((), ())),
                           precision=_HIGH, preferred_element_type=jnp.float32)
    counts = lax.dot_general(onehot, jnp.ones((N, 1), jnp.float32),
                             (((0,), (0,)), ((), ())),
                             precision=_HIGH,
                             preferred_element_type=jnp.float32)
    pooled = sums / jnp.maximum(counts, 1.0)
    out_ref[...] = lax.dot_general(pooled, wout_ref[...],
                                   (((1,), (0,)), ((), ())),
                                   precision=_HIGH,
                                   preferred_element_type=jnp.float32) \
        + bout_ref[...]


def _tc_call(body, out_shape, *args):
    return pl.pallas_call(body, out_shape=out_shape)(*args)


@jax.jit
def kernel(x, edge_index, edge_attr, batch, W1, b1, g1, be1, W2, b2, g2, be2,
           Wout, bout):
    row = edge_index[0]
    col = edge_index[1]
    attr = edge_attr.reshape(E)

    degp = _deg_kernel(col, attr).reshape(2, NPAD, 1)
    u1, dis = _tc_call(
        _tc1_body,
        (jax.ShapeDtypeStruct((N, 64), jnp.float32),
         jax.ShapeDtypeStruct((N, 1), jnp.float32)),
        x, W1, degp)
    s1 = _prop64(row, col, attr, u1)
    u2 = _tc_call(
        _tc2_body, jax.ShapeDtypeStruct((N, 128), jnp.float32),
        s1, u1, dis, b1, g1, be1, W2)
    s2 = _prop128(row, col, attr, u2)
    return _tc_call(
        _tc3_body, jax.ShapeDtypeStruct((G, 1), jnp.float32),
        s2, u2, dis, b2, g2, be2, batch.reshape(N, 1), Wout, bout)


# speedup vs baseline: 10.9851x; 10.9851x over previous
import functools

import jax
import jax.numpy as jnp
from jax import lax
from jax.experimental import pallas as pl
from jax.experimental.pallas import tpu as pltpu
from jax.experimental.pallas import tpu_sc as plsc

N = 10000
E = 320000
F_IN = 128
G = 64
THRESH = 0.5

NPAD = 10240
K = 128
NCHUNK = E // K
NW = 32
CPT = -(-NCHUNK // NW)
ROWS_PER_TILE = NPAD // 16

_MESH = plsc.VectorSubcoreMesh(core_axis_name="c", subcore_axis_name="s")
_HIGH = lax.Precision.HIGHEST


def _worker_id():
    return lax.axis_index("s") * 2 + lax.axis_index("c")


@functools.partial(
    pl.kernel,
    out_type=jax.ShapeDtypeStruct((2, NPAD), jnp.float32),
    mesh=_MESH,
    compiler_params=pltpu.CompilerParams(use_tc_tiling_on_sc=False),
    scratch_types=[
        pltpu.VMEM((K,), jnp.int32),
        pltpu.VMEM((K,), jnp.float32),
        pltpu.VMEM((K,), jnp.float32),
        pltpu.VMEM_SHARED((NPAD,), jnp.float32),
    ],
)
def _deg_kernel(col_hbm, attr_hbm, out_hbm, col_v, w_v, zero_v, acc_sh):
    c = lax.axis_index("c")
    s = lax.axis_index("s")
    wid = _worker_id()

    for j in range(K // 16):
        zero_v[pl.ds(j * 16, 16)] = jnp.zeros((16,), jnp.float32)
    for i in range(ROWS_PER_TILE // K):
        pltpu.sync_copy(zero_v, acc_sh.at[pl.ds(s * ROWS_PER_TILE + i * K, K)])
    plsc.subcore_barrier()

    def body(i, carry):
        chunk = i * NW + wid

        @pl.when(chunk < NCHUNK)
        def _():
            base = chunk * K
            pltpu.sync_copy(col_hbm.at[pl.ds(base, K)], col_v)
            pltpu.sync_copy(attr_hbm.at[pl.ds(base, K)], w_v)
            for j in range(K // 16):
                a = w_v[pl.ds(j * 16, 16)]
                w_v[pl.ds(j * 16, 16)] = jnp.where(a > THRESH, a, 0.0)
            pltpu.sync_copy(w_v, acc_sh.at[col_v], add=True)

        return carry

    lax.fori_loop(0, CPT, body, 0)
    plsc.subcore_barrier()
    pltpu.sync_copy(acc_sh.at[pl.ds(s * ROWS_PER_TILE, ROWS_PER_TILE)],
                    out_hbm.at[c, pl.ds(s * ROWS_PER_TILE, ROWS_PER_TILE)])


def _make_prop(F):
    @functools.partial(
        pl.kernel,
        out_type=jax.ShapeDtypeStruct((2, NPAD, F), jnp.float32),
        mesh=_MESH,
        compiler_params=pltpu.CompilerParams(use_tc_tiling_on_sc=False),
        scratch_types=[
            pltpu.VMEM((K,), jnp.int32),
            pltpu.VMEM((K,), jnp.int32),
            pltpu.VMEM((K,), jnp.float32),
            pltpu.VMEM((K, F), jnp.float32),
            pltpu.VMEM_SHARED((NPAD, F), jnp.float32),
            pltpu.SemaphoreType.DMA,
        ],
    )
    def _prop(row_hbm, col_hbm, attr_hbm, u_hbm, out_hbm,
              row_v, col_v, w_v, msg_v, acc_sh, sem):
        c = lax.axis_index("c")
        s = lax.axis_index("s")
        wid = _worker_id()

        def zrow(j, carry):
            for f in range(F // 16):
                msg_v[j, pl.ds(f * 16, 16)] = jnp.zeros((16,), jnp.float32)
            return carry

        lax.fori_loop(0, K, zrow, 0)
        for i in range(ROWS_PER_TILE // K):
            pltpu.sync_copy(
                msg_v, acc_sh.at[pl.ds(s * ROWS_PER_TILE + i * K, K)])
        plsc.subcore_barrier()

        def body(i, carry):
            chunk = i * NW + wid

            @pl.when(chunk < NCHUNK)
            def _():
                base = chunk * K
                pltpu.sync_copy(row_hbm.at[pl.ds(base, K)], row_v)
                pltpu.sync_copy(col_hbm.at[pl.ds(base, K)], col_v)
                pltpu.sync_copy(attr_hbm.at[pl.ds(base, K)], w_v)
                for j in range(K // 16):
                    a = w_v[pl.ds(j * 16, 16)]
                    w_v[pl.ds(j * 16, 16)] = jnp.where(a > THRESH, a, 0.0)
                pltpu.async_copy(u_hbm.at[row_v], msg_v, sem).wait()

                def scale(g, carry2):
                    wv = w_v[pl.ds(g * 16, 16)]
                    for l in range(16):
                        wsp = jnp.full((16,), wv[l], jnp.float32)
                        r = g * 16 + l
                        for f in range(F // 16):
                            sl = pl.ds(f * 16, 16)
                            msg_v[r, sl] = msg_v[r, sl] * wsp
                    return carry2

                lax.fori_loop(0, K // 16, scale, 0)
                pltpu.sync_copy(msg_v, acc_sh.at[col_v], add=True)

            return carry

        lax.fori_loop(0, CPT, body, 0)
        plsc.subcore_barrier()
        pltpu.sync_copy(
            acc_sh.at[pl.ds(s * ROWS_PER_TILE, ROWS_PER_TILE)],
            out_hbm.at[c, pl.ds(s * ROWS_PER_TILE, ROWS_PER_TILE)])

    return _prop


_prop64 = _make_prop(64)
_prop128 = _make_prop(128)


def _tc1_body(x_ref, w1_ref, degp_ref, u1_ref, dis_ref):
    degp = degp_ref[...]
    deg = degp[0, :N] + degp[1, :N] + 1.0
    dis = jnp.where(deg > 0.0, lax.rsqrt(deg), 0.0)
    xw = lax.dot_general(x_ref[...], w1_ref[...],
                         (((1,), (0,)), ((), ())),
                         precision=_HIGH, preferred_element_type=jnp.float32)
    u1_ref[...] = xw * dis
    dis_ref[...] = dis


def _bn_lrelu(h, g, be):
    mean = jnp.mean(h, axis=0, keepdims=True)
    var = jnp.mean((h - mean) * (h - mean), axis=0, keepdims=True)
    hn = g * (h - mean) / jnp.sqrt(var + 1e-5) + be
    return jnp.where(hn >= 0.0, hn, 0.01 * hn)


def _tc2_body(sp_ref, u1_ref, dis_ref, b1_ref, g1_ref, be1_ref, w2_ref,
              u2_ref):
    sp = sp_ref[...]
    dis = dis_ref[...]
    u1 = u1_ref[...]
    h = dis * (sp[0, :N] + sp[1, :N] + u1) + b1_ref[...]
    h = _bn_lrelu(h, g1_ref[...], be1_ref[...])
    xw = lax.dot_general(h, w2_ref[...], (((1,), (0,)), ((), ())),
                         precision=_HIGH, preferred_element_type=jnp.float32)
    u2_ref[...] = xw * dis


def _tc3_body(sp_ref, u2_ref, dis_ref, b2_ref, g2_ref, be2_ref, batch_ref,
              wout_ref, bout_ref, out_ref):
    sp = sp_ref[...]
    dis = dis_ref[...]
    u2 = u2_ref[...]
    h = dis * (sp[0, :N] + sp[1, :N] + u2) + b2_ref[...]
    h = _bn_lrelu(h, g2_ref[...], be2_ref[...])
    onehot = (batch_ref[...] ==
              lax.broadcasted_iota(jnp.int32, (1, G), 1)).astype(jnp.float32)
    sums = lax.dot_general(onehot, h, (((0,), (0,)), ((), ())),
                           precision=_HIGH, preferred_element_type=jnp.float32)
    counts = lax.dot_general(onehot, jnp.ones((N, 1), jnp.float32),
                             (((0,), (0,)), ((), ())),
                             precision=_HIGH,
                             preferred_element_type=jnp.float32)
    pooled = sums / jnp.maximum(counts, 1.0)
    out_ref[...] = lax.dot_general(pooled, wout_ref[...],
                                   (((1,), (0,)), ((), ())),
                                   precision=_HIGH,
                                   preferred_element_type=jnp.float32) \
        + bout_ref[...]


def _tc_call(body, out_shape, *args):
    return pl.pallas_call(body, out_shape=out_shape)(*args)


@jax.jit
def kernel(x, edge_index, edge_attr, batch, W1, b1, g1, be1, W2, b2, g2, be2,
           Wout, bout):
    row = edge_index[0]
    col = edge_index[1]
    attr = edge_attr.reshape(E)

    degp = _deg_kernel(col, attr).reshape(2, NPAD, 1)
    u1, dis = _tc_call(
        _tc1_body,
        (jax.ShapeDtypeStruct((N, 64), jnp.float32),
         jax.ShapeDtypeStruct((N, 1), jnp.float32)),
        x, W1, degp)
    s1 = _prop64(row, col, attr, u1)
    u2 = _tc_call(
        _tc2_body, jax.ShapeDtypeStruct((N, 128), jnp.float32),
        s1, u1, dis, b1, g1, be1, W2)
    s2 = _prop128(row, col, attr, u2)
    return _tc_call(
        _tc3_body, jax.ShapeDtypeStruct((G, 1), jnp.float32),
        s2, u2, dis, b2, g2, be2, batch.reshape(N, 1), Wout, bout)
